# baseline (speedup 1.0000x reference)
import jax
import jax.numpy as jnp
import numpy as np
from jax import lax
from jax.experimental import pallas as pl
from jax.experimental.pallas import tpu as pltpu
from jax.experimental.pallas import tpu_sc as plsc

BATCH_NUM = 1024
WIN_SIZE = 50
EMBED_DIM = 64
N = 102400

NC = 2
NS = 16
L = 16
PASSES = 2

Q_B = BATCH_NUM // (NC * PASSES)
Q_S = Q_B * WIN_SIZE
TRASH = Q_S
ACC_ROWS = 14336
ZROWS_PER_SUB = ACC_ROWS // NS
PER_SUB = N // NS
CHUNK = 128
NCHUNK = PER_SUB // CHUNK
B_PER_SUB = Q_B // NS
OUT_ROW = WIN_SIZE * EMBED_DIM


def _body(ids_hbm, bat_hbm, win_hbm, table_hbm, scat_hbm, out_hbm,
          ids2d, bat2d, win2d, lseg2d, rows_a, rows_b, ones_v,
          scat_v, buf2d, tbuf, cbuf, invbuf,
          acc_s, cnt_s, sem_a, sem_b):
  c = lax.axis_index("c")
  s = lax.axis_index("s")

  zf = jnp.zeros((L,), jnp.float32)

  def zrow(i, _):
    def zcol(k, _):
      rows_a[i, pl.ds(k * L, L)] = zf
      return 0
    return lax.fori_loop(0, EMBED_DIM // L, zcol, 0)
  lax.fori_loop(0, CHUNK, zrow, 0)

  def zone(k, _):
    ones_v[pl.ds(k * L, L)] = zf
    return 0
  lax.fori_loop(0, CHUNK // L, zone, 0)

  pltpu.sync_copy(ids_hbm.at[s], ids2d)
  pltpu.sync_copy(bat_hbm.at[s], bat2d)
  pltpu.sync_copy(win_hbm.at[s], win2d)
  pltpu.sync_copy(scat_hbm, scat_v)

  zbase = s * ZROWS_PER_SUB

  for p in range(PASSES):
    q_base = (c * PASSES + p) * Q_S

    def zcp(i, _):
      pltpu.sync_copy(rows_a, acc_s.at[pl.ds(zbase + i * CHUNK, CHUNK)])
      pltpu.sync_copy(ones_v, cnt_s.at[pl.ds(zbase + i * CHUNK, CHUNK)])
      return 0
    lax.fori_loop(0, ZROWS_PER_SUB // CHUNK, zcp, 0)

    def lseg_outer(j, _):
      def lseg_inner(i, _):
        b16 = bat2d[j, pl.ds(i * L, L)]
        w16 = win2d[j, pl.ds(i * L, L)]
        seg = b16 * WIN_SIZE + w16 - q_base
        inb = (seg >= 0) & (seg < Q_S)
        lseg2d[j, pl.ds(i * L, L)] = jnp.where(inb, seg, TRASH)
        return 0
      return lax.fori_loop(0, CHUNK // L, lseg_inner, 0)
    lax.fori_loop(0, NCHUNK, lseg_outer, 0)

    of = jnp.ones((L,), jnp.float32)

    def fone(k, _):
      ones_v[pl.ds(k * L, L)] = of
      return 0
    lax.fori_loop(0, CHUNK // L, fone, 0)

    plsc.subcore_barrier()

    pltpu.async_copy(table_hbm.at[ids2d.at[0]], rows_a, sem_a)

    def pp(jj, _):
      j0 = 2 * jj
      pltpu.async_copy(table_hbm.at[ids2d.at[j0 + 1]], rows_b, sem_b)
      pltpu.make_async_copy(table_hbm.at[ids2d.at[0]], rows_a, sem_a).wait()
      pltpu.sync_copy(rows_a, acc_s.at[lseg2d.at[j0]], add=True)
      pltpu.sync_copy(ones_v, cnt_s.at[lseg2d.at[j0]], add=True)
      jn = jnp.minimum(j0 + 2, NCHUNK - 1)
      pltpu.async_copy(table_hbm.at[ids2d.at[jn]], rows_a, sem_a)
      pltpu.make_async_copy(table_hbm.at[ids2d.at[0]], rows_b, sem_b).wait()
      pltpu.sync_copy(rows_b, acc_s.at[lseg2d.at[j0 + 1]], add=True)
      pltpu.sync_copy(ones_v, cnt_s.at[lseg2d.at[j0 + 1]], add=True)
      return 0
    lax.fori_loop(0, NCHUNK // 2, pp, 0)
    pltpu.make_async_copy(table_hbm.at[ids2d.at[0]], rows_a, sem_a).wait()

    plsc.subcore_barrier()

    def wout(i, _):
      lb = s * B_PER_SUB + i
      t0 = lb * WIN_SIZE
      base8 = pl.multiple_of(t0 - (t0 & 7), 8)
      off = t0 - base8
      pltpu.sync_copy(acc_s.at[pl.ds(t0, WIN_SIZE)], buf2d)
      pltpu.sync_copy(cnt_s.at[pl.ds(base8, CHUNK)], cbuf)

      def inv_k(k, _):
        cc = cbuf[pl.ds(k * L, L)]
        invbuf[pl.ds(k * L, L)] = jnp.where(cc > 0, 1.0 / cc, 0.0)
        return 0
      lax.fori_loop(0, CHUNK // L, inv_k, 0)

      def tr_w(w, _):
        wsplat = scat_v[pl.ds(0, L)] * 0 + (w + off)
        iv = plsc.load_gather(invbuf, [wsplat])

        def tr_k(k, _):
          v16 = buf2d[w, pl.ds(k * L, L)]
          sidx = scat_v[pl.ds((w * (EMBED_DIM // L) + k) * L, L)]
          plsc.store_scatter(tbuf, [sidx], v16 * iv)
          return 0
        return lax.fori_loop(0, EMBED_DIM // L, tr_k, 0)
      lax.fori_loop(0, WIN_SIZE, tr_w, 0)

      b = (c * PASSES + p) * Q_B + lb
      pltpu.sync_copy(tbuf, out_hbm.at[b])
      return 0
    lax.fori_loop(0, B_PER_SUB, wout, 0)

    if p + 1 < PASSES:
      def rezero(k, _):
        ones_v[pl.ds(k * L, L)] = zf
        return 0
      lax.fori_loop(0, CHUNK // L, rezero, 0)

      def rezrow(i, _):
        def rezcol(k, _):
          rows_a[i, pl.ds(k * L, L)] = zf
          return 0
        return lax.fori_loop(0, EMBED_DIM // L, rezcol, 0)
      lax.fori_loop(0, CHUNK, rezrow, 0)

      plsc.subcore_barrier()


_sc_call = pl.kernel(
    _body,
    out_type=jax.ShapeDtypeStruct((BATCH_NUM, OUT_ROW), jnp.float32),
    mesh=plsc.VectorSubcoreMesh(core_axis_name="c", subcore_axis_name="s"),
    compiler_params=pltpu.CompilerParams(
        needs_layout_passes=False, use_tc_tiling_on_sc=False),
    scratch_types=[
        pltpu.VMEM((NCHUNK, CHUNK), jnp.int32),
        pltpu.VMEM((NCHUNK, CHUNK), jnp.int32),
        pltpu.VMEM((NCHUNK, CHUNK), jnp.int32),
        pltpu.VMEM((NCHUNK, CHUNK), jnp.int32),
        pltpu.VMEM((CHUNK, EMBED_DIM), jnp.float32),
        pltpu.VMEM((CHUNK, EMBED_DIM), jnp.float32),
        pltpu.VMEM((CHUNK,), jnp.float32),
        pltpu.VMEM((OUT_ROW,), jnp.int32),
        pltpu.VMEM((WIN_SIZE, EMBED_DIM), jnp.float32),
        pltpu.VMEM((OUT_ROW,), jnp.float32),
        pltpu.VMEM((CHUNK,), jnp.float32),
        pltpu.VMEM((CHUNK,), jnp.float32),
        pltpu.VMEM_SHARED((ACC_ROWS, EMBED_DIM), jnp.float32),
        pltpu.VMEM_SHARED((ACC_ROWS,), jnp.float32),
        pltpu.SemaphoreType.DMA,
        pltpu.SemaphoreType.DMA,
    ],
)

_w = np.arange(WIN_SIZE, dtype=np.int32)[:, None]
_d = np.arange(EMBED_DIM, dtype=np.int32)[None, :]
_SCAT = (_d * WIN_SIZE + _w).reshape(OUT_ROW)


@jax.jit
def kernel(input, batch_i, win_i, table):
  ids3 = input.reshape(NS, NCHUNK, CHUNK)
  bat3 = batch_i.reshape(NS, NCHUNK, CHUNK)
  win3 = win_i.reshape(NS, NCHUNK, CHUNK)
  out = _sc_call(ids3, bat3, win3, table, jnp.asarray(_SCAT))
  return out.reshape(BATCH_NUM, EMBED_DIM, WIN_SIZE)
